# baseline (speedup 1.0000x reference)
import functools

import jax
import jax.numpy as jnp
from jax import lax
from jax.experimental import pallas as pl
from jax.experimental.pallas import tpu as pltpu
from jax.experimental.pallas import tpu_sc as plsc

TOPK = 2
NE = 64
D = 768

NW = 32
L = 16


def _softmax_body(x_ref, w_ref, pf_ref):
    x = x_ref[0]
    w = w_ref[...]
    logits = jax.lax.dot_general(
        x, w, (((1,), (1,)), ((), ())), preferred_element_type=jnp.float32)
    mx = jnp.max(logits, axis=-1, keepdims=True)
    e = jnp.exp(logits - mx)
    s = jnp.sum(e, axis=-1, keepdims=True)
    pf_ref[0] = e * (1.0 / s)


def _tc_softmax(inputs, W):
    B, S, _ = inputs.shape
    BT = 4096
    NB = S // BT
    return pl.pallas_call(
        _softmax_body,
        grid=(B, NB),
        in_specs=[
            pl.BlockSpec((1, BT, D), lambda b, i: (b, i, 0)),
            pl.BlockSpec((NE, D), lambda b, i: (0, 0)),
        ],
        out_specs=pl.BlockSpec((1, BT, NE), lambda b, i: (b, i, 0)),
        out_shape=jax.ShapeDtypeStruct((B, S, NE), jnp.float32),
    )(inputs, W)


def _make_sc_topk(B, S, CH):
    TPW = B * S // NW
    NCH = TPW // CH
    NG = CH // L
    WPB = S // TPW

    mesh = plsc.VectorSubcoreMesh(core_axis_name="c", subcore_axis_name="s")

    @functools.partial(
        pl.kernel,
        out_type=[
            jax.ShapeDtypeStruct((B, S, NE), jnp.float32),
            jax.ShapeDtypeStruct((B, S, TOPK), jnp.int32),
            jax.ShapeDtypeStruct((B, S, NE), jnp.float32),
        ],
        mesh=mesh,
        compiler_params=pltpu.CompilerParams(needs_layout_passes=False),
        scratch_types=[
            pltpu.VMEM((CH, NE), jnp.float32),
            pltpu.VMEM((CH, NE), jnp.float32),
            pltpu.VMEM((CH, NE), jnp.float32),
            pltpu.VMEM((CH, TOPK), jnp.int32),
        ],
    )
    def sc_topk(pf_hbm, mask_hbm, idx_hbm, rp_hbm, probs_v, mask_v, rp_v, idx_v):
        wid = lax.axis_index("c") * 16 + lax.axis_index("s")
        b = wid // WPB
        s_base = (wid % WPB) * TPW
        iota = lax.iota(jnp.int32, L)
        zero_i = jnp.zeros((L,), jnp.int32)
        one_i = jnp.ones((L,), jnp.int32)
        ones_f = jnp.ones((L,), jnp.float32)
        zeros_f = jnp.zeros((L,), jnp.float32)

        def zrow(r, _):
            for k in range(NE // L):
                mask_v[r, pl.ds(k * L, L)] = zeros_f
                rp_v[r, pl.ds(k * L, L)] = zeros_f
            return 0
        lax.fori_loop(0, CH, zrow, 0)

        def chunk_body(ci, _):
            s_off = s_base + ci * CH
            pltpu.sync_copy(pf_hbm.at[b, pl.ds(s_off, CH)], probs_v)

            def group_body(g, _):
                rows = g * L + iota
                m1 = jnp.full((L,), -1.0, jnp.float32)
                m2 = jnp.full((L,), -1.0, jnp.float32)
                i1 = zero_i
                i2 = zero_i
                for j in range(NE):
                    jv = jnp.full((L,), j, jnp.int32)
                    v = plsc.load_gather(probs_v, [rows, jv])
                    gt1 = v > m1
                    gt2 = v > m2
                    i2 = jnp.where(gt1, i1, jnp.where(gt2, jv, i2))
                    m2 = jnp.where(gt1, m1, jnp.where(gt2, v, m2))
                    i1 = jnp.where(gt1, jv, i1)
                    m1 = jnp.where(gt1, v, m1)
                plsc.store_scatter(mask_v, [rows, i1], ones_f)
                plsc.store_scatter(mask_v, [rows, i2], ones_f)
                denom = m1 + m2
                plsc.store_scatter(rp_v, [rows, i1], m1 / denom)
                plsc.store_scatter(rp_v, [rows, i2], m2 / denom)
                plsc.store_scatter(idx_v, [rows, zero_i], i1)
                plsc.store_scatter(idx_v, [rows, one_i], i2)
                return 0
            lax.fori_loop(0, NG, group_body, 0)

            pltpu.sync_copy(mask_v, mask_hbm.at[b, pl.ds(s_off, CH)])
            pltpu.sync_copy(rp_v, rp_hbm.at[b, pl.ds(s_off, CH)])
            pltpu.sync_copy(idx_v, idx_hbm.at[b, pl.ds(s_off, CH)])

            def rezero(g, _):
                rows = g * L + iota
                i1 = plsc.load_gather(idx_v, [rows, zero_i])
                i2 = plsc.load_gather(idx_v, [rows, one_i])
                plsc.store_scatter(mask_v, [rows, i1], zeros_f)
                plsc.store_scatter(mask_v, [rows, i2], zeros_f)
                plsc.store_scatter(rp_v, [rows, i1], zeros_f)
                plsc.store_scatter(rp_v, [rows, i2], zeros_f)
                return 0
            lax.fori_loop(0, NG, rezero, 0)
            return 0
        lax.fori_loop(0, NCH, chunk_body, 0)

    return sc_topk


def kernel(inputs, W):
    B, S, _ = inputs.shape
    pf = _tc_softmax(inputs, W)
    mask, idx, rp = _make_sc_topk(B, S, CH=128)(pf)
    return (mask, idx, rp, pf)


# speedup vs baseline: 1.0217x; 1.0217x over previous
import functools

import jax
import jax.numpy as jnp
from jax import lax
from jax.experimental import pallas as pl
from jax.experimental.pallas import tpu as pltpu
from jax.experimental.pallas import tpu_sc as plsc

TOPK = 2
NE = 64
D = 768

NW = 32
L = 16


def _softmax_body(x_ref, w_ref, pf_ref):
    x = x_ref[0]
    w = w_ref[...]
    logits = jax.lax.dot_general(
        x, w, (((1,), (1,)), ((), ())), preferred_element_type=jnp.float32)
    mx = jnp.max(logits, axis=-1, keepdims=True)
    e = jnp.exp(logits - mx)
    s = jnp.sum(e, axis=-1, keepdims=True)
    pf_ref[0] = e * (1.0 / s)


def _tc_softmax(inputs, W):
    B, S, _ = inputs.shape
    BT = 4096
    NB = S // BT
    return pl.pallas_call(
        _softmax_body,
        grid=(B, NB),
        in_specs=[
            pl.BlockSpec((1, BT, D), lambda b, i: (b, i, 0)),
            pl.BlockSpec((NE, D), lambda b, i: (0, 0)),
        ],
        out_specs=pl.BlockSpec((1, BT, NE), lambda b, i: (b, i, 0)),
        out_shape=jax.ShapeDtypeStruct((B, S, NE), jnp.float32),
    )(inputs, W)


def _make_sc_topk(B, S, CH):
    TPW = B * S // NW
    NCH = TPW // CH
    NG = CH // L
    WPB = S // TPW

    mesh = plsc.VectorSubcoreMesh(core_axis_name="c", subcore_axis_name="s")

    @functools.partial(
        pl.kernel,
        out_type=[
            jax.ShapeDtypeStruct((B, S, NE), jnp.float32),
            jax.ShapeDtypeStruct((B, S, TOPK), jnp.int32),
            jax.ShapeDtypeStruct((B, S, NE), jnp.float32),
        ],
        mesh=mesh,
        compiler_params=pltpu.CompilerParams(needs_layout_passes=False),
        scratch_types=[
            pltpu.VMEM((CH, NE), jnp.float32),
            pltpu.VMEM((CH, NE), jnp.float32),
            pltpu.VMEM((CH, NE), jnp.float32),
            pltpu.VMEM((CH, TOPK), jnp.int32),
        ],
    )
    def sc_topk(pf_hbm, mask_hbm, idx_hbm, rp_hbm, probs_v, mask_v, rp_v, idx_v):
        wid = lax.axis_index("c") * 16 + lax.axis_index("s")
        b = wid // WPB
        s_base = (wid % WPB) * TPW
        iota = lax.iota(jnp.int32, L)
        zero_i = jnp.zeros((L,), jnp.int32)
        one_i = jnp.ones((L,), jnp.int32)
        ones_f = jnp.ones((L,), jnp.float32)
        zeros_f = jnp.zeros((L,), jnp.float32)

        def zrow(r, _):
            for k in range(NE // L):
                mask_v[r, pl.ds(k * L, L)] = zeros_f
                rp_v[r, pl.ds(k * L, L)] = zeros_f
            return 0
        lax.fori_loop(0, CH, zrow, 0)

        def chunk_body(ci, _):
            s_off = s_base + ci * CH
            pltpu.sync_copy(pf_hbm.at[b, pl.ds(s_off, CH)], probs_v)

            def group_body(g, _):
                rows = g * L + iota
                NBLK = 4
                accs = []
                for k in range(NBLK):
                    m1 = jnp.full((L,), -1.0, jnp.float32)
                    m2 = jnp.full((L,), -1.0, jnp.float32)
                    i1 = zero_i
                    i2 = zero_i
                    for j in range(k * (NE // NBLK), (k + 1) * (NE // NBLK)):
                        jv = jnp.full((L,), j, jnp.int32)
                        v = plsc.load_gather(probs_v, [rows, jv])
                        gt1 = v > m1
                        gt2 = v > m2
                        i2 = jnp.where(gt1, i1, jnp.where(gt2, jv, i2))
                        m2 = jnp.where(gt1, m1, jnp.where(gt2, v, m2))
                        i1 = jnp.where(gt1, jv, i1)
                        m1 = jnp.where(gt1, v, m1)
                    accs.append((m1, m2, i1, i2))

                def merge(a, bb):
                    am1, am2, ai1, ai2 = a
                    bm1, bm2, bi1, bi2 = bb
                    gt = bm1 > am1
                    M1 = jnp.where(gt, bm1, am1)
                    I1 = jnp.where(gt, bi1, ai1)
                    c1 = jnp.where(gt, am1, bm1)
                    ci1 = jnp.where(gt, ai1, bi1)
                    c2 = jnp.where(gt, bm2, am2)
                    ci2 = jnp.where(gt, bi2, ai2)
                    gt2 = c2 > c1
                    M2 = jnp.where(gt2, c2, c1)
                    I2 = jnp.where(gt2, ci2, ci1)
                    return (M1, M2, I1, I2)

                t01 = merge(accs[0], accs[1])
                t23 = merge(accs[2], accs[3])
                m1, m2, i1, i2 = merge(t01, t23)
                plsc.store_scatter(mask_v, [rows, i1], ones_f)
                plsc.store_scatter(mask_v, [rows, i2], ones_f)
                denom = m1 + m2
                plsc.store_scatter(rp_v, [rows, i1], m1 / denom)
                plsc.store_scatter(rp_v, [rows, i2], m2 / denom)
                plsc.store_scatter(idx_v, [rows, zero_i], i1)
                plsc.store_scatter(idx_v, [rows, one_i], i2)
                return 0
            lax.fori_loop(0, NG, group_body, 0)

            pltpu.sync_copy(mask_v, mask_hbm.at[b, pl.ds(s_off, CH)])
            pltpu.sync_copy(rp_v, rp_hbm.at[b, pl.ds(s_off, CH)])
            pltpu.sync_copy(idx_v, idx_hbm.at[b, pl.ds(s_off, CH)])

            def rezero(g, _):
                rows = g * L + iota
                i1 = plsc.load_gather(idx_v, [rows, zero_i])
                i2 = plsc.load_gather(idx_v, [rows, one_i])
                plsc.store_scatter(mask_v, [rows, i1], zeros_f)
                plsc.store_scatter(mask_v, [rows, i2], zeros_f)
                plsc.store_scatter(rp_v, [rows, i1], zeros_f)
                plsc.store_scatter(rp_v, [rows, i2], zeros_f)
                return 0
            lax.fori_loop(0, NG, rezero, 0)
            return 0
        lax.fori_loop(0, NCH, chunk_body, 0)

    return sc_topk


def kernel(inputs, W):
    B, S, _ = inputs.shape
    pf = _tc_softmax(inputs, W)
    mask, idx, rp = _make_sc_topk(B, S, CH=256)(pf)
    return (mask, idx, rp, pf)
